# baseline (speedup 1.0000x reference)
import functools
import math

import jax
import jax.numpy as jnp
from jax.experimental import pallas as pl
from jax.experimental.pallas import tpu as pltpu

S = 2048
D = 1024
FF = 4096
E = 6
V = 4
K = 2
FIXED = E - V
LANES = 128
FF_CHUNK = 1024
NFF = FF // FF_CHUNK
_INV_SQRT2 = 0.7071067811865476


def _router_kernel(x_ref, wr_ref, w_ref, aux_ref):
    xs = x_ref[...]
    logits = jax.lax.dot_general(
        xs, wr_ref[...].astype(jnp.bfloat16),
        (((1,), (0,)), ((), ())),
        preferred_element_type=jnp.float32)
    lane = jax.lax.broadcasted_iota(jnp.int32, (S, LANES), 1)
    valid = lane < V
    neg = jnp.float32(-1e30)
    logits = jnp.where(valid, logits, neg)
    m = jnp.max(logits, axis=1, keepdims=True)
    ex = jnp.where(valid, jnp.exp(logits - m), 0.0)
    denom = jnp.sum(ex, axis=1, keepdims=True)
    probs = ex / denom
    p1 = jnp.max(probs, axis=1, keepdims=True)
    big = jnp.int32(LANES)
    i1 = jnp.min(jnp.where((probs == p1) & valid, lane, big), axis=1,
                 keepdims=True)
    rest = jnp.where(lane == i1, neg, probs)
    p2 = jnp.max(rest, axis=1, keepdims=True)
    i2 = jnp.min(jnp.where((rest == p2) & valid, lane, big), axis=1,
                 keepdims=True)
    wsum = p1 + p2
    w1 = p1 / wsum
    w2 = p2 / wsum
    sel1 = lane == i1
    sel2 = lane == i2
    w_ref[...] = jnp.where(sel1, w1, 0.0) + jnp.where(sel2, w2, 0.0)
    counts = jnp.sum(sel1.astype(jnp.float32) + sel2.astype(jnp.float32),
                     axis=0, keepdims=True)
    psum = jnp.sum(probs, axis=0, keepdims=True)
    density = psum / jnp.float32(S)
    usage = counts / jnp.float32(S)
    balance = jnp.sum(density * usage) * jnp.float32(E)
    important = jnp.sum(psum * psum) / jnp.float32(E)
    aux_ref[0, 0] = balance + important


def _moe_kernel(x_ref, w1_ref, b1_ref, w2_ref, b2_ref, w_ref, g_ref, bt_ref,
                y_ref, acc_ref):
    e = pl.program_id(0)
    f = pl.program_id(1)

    @pl.when((e == 0) & (f == 0))
    def _():
        acc_ref[...] = jnp.zeros_like(acc_ref)

    xb = x_ref[...]
    w1c = w1_ref[0].astype(jnp.bfloat16)
    h = jax.lax.dot_general(xb, w1c, (((1,), (0,)), ((), ())),
                            preferred_element_type=jnp.float32)
    h = h + b1_ref[pl.ds(e, 1), pl.ds(f * FF_CHUNK, FF_CHUNK)]
    h = 0.5 * h * (1.0 + jax.lax.erf(h * _INV_SQRT2))

    lane = jax.lax.broadcasted_iota(jnp.int32, (S, LANES), 1)
    wsel = jnp.sum(jnp.where(lane == e - FIXED, w_ref[...], 0.0), axis=1,
                   keepdims=True)
    wcol = jnp.where(e < FIXED, 1.0, wsel)
    hw = (h * wcol).astype(jnp.bfloat16)
    w2c = w2_ref[0].astype(jnp.bfloat16)
    acc_ref[...] += jax.lax.dot_general(hw, w2c, (((1,), (0,)), ((), ())),
                                        preferred_element_type=jnp.float32)

    @pl.when(f == 0)
    def _():
        acc_ref[...] += wcol * b2_ref[pl.ds(e, 1), :]

    @pl.when((e == E - 1) & (f == NFF - 1))
    def _():
        acc = acc_ref[...]
        mu = jnp.mean(acc, axis=1, keepdims=True)
        var = jnp.mean((acc - mu) ** 2, axis=1, keepdims=True)
        y_ref[...] = ((acc - mu) * jax.lax.rsqrt(var + 1e-5) * g_ref[...]
                      + bt_ref[...])


@jax.jit
def kernel(x, Wr, W1, b1, W2, b2, gamma, beta):
    xs = x.reshape(S, D).astype(jnp.bfloat16)
    wr_pad = jnp.zeros((D, LANES), jnp.float32).at[:, :V].set(Wr)

    w_var, aux = pl.pallas_call(
        _router_kernel,
        out_shape=[
            jax.ShapeDtypeStruct((S, LANES), jnp.float32),
            jax.ShapeDtypeStruct((1, 1), jnp.float32),
        ],
        in_specs=[
            pl.BlockSpec((S, D), lambda: (0, 0)),
            pl.BlockSpec((D, LANES), lambda: (0, 0)),
        ],
        out_specs=[
            pl.BlockSpec((S, LANES), lambda: (0, 0)),
            pl.BlockSpec(memory_space=pltpu.SMEM),
        ],
    )(xs, wr_pad)

    y = pl.pallas_call(
        _moe_kernel,
        grid=(E, NFF),
        out_shape=jax.ShapeDtypeStruct((S, D), jnp.float32),
        in_specs=[
            pl.BlockSpec((S, D), lambda e, f: (0, 0)),
            pl.BlockSpec((1, D, FF_CHUNK), lambda e, f: (e, 0, f)),
            pl.BlockSpec((E, FF), lambda e, f: (0, 0)),
            pl.BlockSpec((1, FF_CHUNK, D), lambda e, f: (e, f, 0)),
            pl.BlockSpec((E, D), lambda e, f: (0, 0)),
            pl.BlockSpec((S, LANES), lambda e, f: (0, 0)),
            pl.BlockSpec((1, D), lambda e, f: (0, 0)),
            pl.BlockSpec((1, D), lambda e, f: (0, 0)),
        ],
        out_specs=pl.BlockSpec((S, D), lambda e, f: (0, 0)),
        scratch_shapes=[pltpu.VMEM((S, D), jnp.float32)],
    )(xs, W1, b1, W2, b2, w_var, gamma.reshape(1, D),
      beta.reshape(1, D))

    return y.reshape(1, S, D), aux[0, 0]


# speedup vs baseline: 1.0449x; 1.0449x over previous
import functools
import math

import jax
import jax.numpy as jnp
from jax.experimental import pallas as pl
from jax.experimental.pallas import tpu as pltpu

S = 2048
D = 1024
FF = 4096
E = 6
V = 4
K = 2
FIXED = E - V
LANES = 128
FF_CHUNK = 1024
NFF = FF // FF_CHUNK
_INV_SQRT2 = 0.7071067811865476


def _router_kernel(x_ref, wr_ref, w_ref, aux_ref):
    xs = x_ref[...]
    logits = jax.lax.dot_general(
        xs, wr_ref[...].astype(jnp.bfloat16),
        (((1,), (0,)), ((), ())),
        preferred_element_type=jnp.float32)
    lane = jax.lax.broadcasted_iota(jnp.int32, (S, LANES), 1)
    valid = lane < V
    neg = jnp.float32(-1e30)
    logits = jnp.where(valid, logits, neg)
    m = jnp.max(logits, axis=1, keepdims=True)
    ex = jnp.where(valid, jnp.exp(logits - m), 0.0)
    denom = jnp.sum(ex, axis=1, keepdims=True)
    probs = ex / denom
    p1 = jnp.max(probs, axis=1, keepdims=True)
    big = jnp.int32(LANES)
    i1 = jnp.min(jnp.where((probs == p1) & valid, lane, big), axis=1,
                 keepdims=True)
    rest = jnp.where(lane == i1, neg, probs)
    p2 = jnp.max(rest, axis=1, keepdims=True)
    i2 = jnp.min(jnp.where((rest == p2) & valid, lane, big), axis=1,
                 keepdims=True)
    wsum = p1 + p2
    w1 = p1 / wsum
    w2 = p2 / wsum
    sel1 = lane == i1
    sel2 = lane == i2
    w_ref[...] = jnp.where(sel1, w1, 0.0) + jnp.where(sel2, w2, 0.0)
    counts = jnp.sum(sel1.astype(jnp.float32) + sel2.astype(jnp.float32),
                     axis=0, keepdims=True)
    psum = jnp.sum(probs, axis=0, keepdims=True)
    density = psum / jnp.float32(S)
    usage = counts / jnp.float32(S)
    balance = jnp.sum(density * usage) * jnp.float32(E)
    important = jnp.sum(psum * psum) / jnp.float32(E)
    aux_ref[0, 0] = balance + important


def _moe_kernel(x_ref, w1_ref, b1_ref, w2_ref, b2_ref, w_ref, g_ref, bt_ref,
                y_ref, acc_ref):
    e = pl.program_id(0)
    f = pl.program_id(1)

    @pl.when((e == 0) & (f == 0))
    def _():
        acc_ref[...] = jnp.zeros_like(acc_ref)

    xb = x_ref[...]
    w1c = w1_ref[0].astype(jnp.bfloat16)
    h = jax.lax.dot_general(xb, w1c, (((1,), (0,)), ((), ())),
                            preferred_element_type=jnp.float32
                            ).astype(jnp.bfloat16)
    h = h + b1_ref[pl.ds(e, 1), pl.ds(f * FF_CHUNK, FF_CHUNK)].astype(
        jnp.bfloat16)
    h = (jnp.bfloat16(0.5) * h
         * (jnp.bfloat16(1.0)
            + jax.lax.erf(h * jnp.bfloat16(_INV_SQRT2))))

    lane = jax.lax.broadcasted_iota(jnp.int32, (S, LANES), 1)
    wsel = jnp.sum(jnp.where(lane == e - FIXED, w_ref[...], 0.0), axis=1,
                   keepdims=True)
    wcol = jnp.where(e < FIXED, 1.0, wsel)
    hw = h * wcol.astype(jnp.bfloat16)
    w2c = w2_ref[0].astype(jnp.bfloat16)
    acc_ref[...] += jax.lax.dot_general(hw, w2c, (((1,), (0,)), ((), ())),
                                        preferred_element_type=jnp.float32)

    @pl.when(f == 0)
    def _():
        acc_ref[...] += wcol * b2_ref[pl.ds(e, 1), :]

    @pl.when((e == E - 1) & (f == NFF - 1))
    def _():
        acc = acc_ref[...]
        mu = jnp.mean(acc, axis=1, keepdims=True)
        var = jnp.mean((acc - mu) ** 2, axis=1, keepdims=True)
        y_ref[...] = ((acc - mu) * jax.lax.rsqrt(var + 1e-5) * g_ref[...]
                      + bt_ref[...])


@jax.jit
def kernel(x, Wr, W1, b1, W2, b2, gamma, beta):
    xs = x.reshape(S, D).astype(jnp.bfloat16)
    wr_pad = jnp.zeros((D, LANES), jnp.float32).at[:, :V].set(Wr)

    w_var, aux = pl.pallas_call(
        _router_kernel,
        out_shape=[
            jax.ShapeDtypeStruct((S, LANES), jnp.float32),
            jax.ShapeDtypeStruct((1, 1), jnp.float32),
        ],
        in_specs=[
            pl.BlockSpec((S, D), lambda: (0, 0)),
            pl.BlockSpec((D, LANES), lambda: (0, 0)),
        ],
        out_specs=[
            pl.BlockSpec((S, LANES), lambda: (0, 0)),
            pl.BlockSpec(memory_space=pltpu.SMEM),
        ],
    )(xs, wr_pad)

    y = pl.pallas_call(
        _moe_kernel,
        grid=(E, NFF),
        out_shape=jax.ShapeDtypeStruct((S, D), jnp.float32),
        in_specs=[
            pl.BlockSpec((S, D), lambda e, f: (0, 0)),
            pl.BlockSpec((1, D, FF_CHUNK), lambda e, f: (e, 0, f)),
            pl.BlockSpec((E, FF), lambda e, f: (0, 0)),
            pl.BlockSpec((1, FF_CHUNK, D), lambda e, f: (e, f, 0)),
            pl.BlockSpec((E, D), lambda e, f: (0, 0)),
            pl.BlockSpec((S, LANES), lambda e, f: (0, 0)),
            pl.BlockSpec((1, D), lambda e, f: (0, 0)),
            pl.BlockSpec((1, D), lambda e, f: (0, 0)),
        ],
        out_specs=pl.BlockSpec((S, D), lambda e, f: (0, 0)),
        scratch_shapes=[pltpu.VMEM((S, D), jnp.float32)],
    )(xs, W1, b1, W2, b2, w_var, gamma.reshape(1, D),
      beta.reshape(1, D))

    return y.reshape(1, S, D), aux[0, 0]
